# baseline (speedup 1.0000x reference)
import functools

import jax
import jax.numpy as jnp
from jax import lax
from jax.experimental import pallas as pl
from jax.experimental.pallas import tpu as pltpu
from jax.experimental.pallas import tpu_sc as plsc

_LANES = 16


def _full16(x):
    return jnp.full((_LANES,), x, dtype=jnp.int32)


def _make_tc_repack(vocab, dim):
    chunk = 512
    n_chunks = -(-vocab // chunk)
    x_rows = n_chunks * 128

    def body(t_ref, x_ref):
        t512 = jnp.swapaxes(t_ref[...], 0, 1)
        t3 = jnp.reshape(t512, (128, 128 // dim, dim))
        for q in range(128 // dim):
            x_ref[:, pl.ds(dim * q, dim)] = t3[:, q, :]

    repack = pl.pallas_call(
        body,
        grid=(n_chunks,),
        in_specs=[pl.BlockSpec((dim, chunk), lambda i: (0, i))],
        out_specs=pl.BlockSpec((128, 128), lambda i: (i, 0)),
        out_shape=jax.ShapeDtypeStruct((x_rows, 128), jnp.float32),
    )
    return repack, x_rows


def _make_repack(vocab, dim):
    info = plsc.get_sparse_core_info()
    num_cores, num_subcores = info.num_cores, info.num_subcores
    num_workers = num_cores * num_subcores
    n_blocks = (vocab + 127) // 128
    n_slots = 2 * -(-(-(-n_blocks // num_workers) * num_workers) // (2 * num_workers))
    slots_per_tile = -(-n_blocks // num_workers)
    if slots_per_tile % 2:
        slots_per_tile += 1
    n_extra = slots_per_tile * num_workers - n_blocks
    x_rows = 32 * (n_blocks + n_extra)
    mesh = plsc.VectorSubcoreMesh(core_axis_name="c", subcore_axis_name="s")

    @functools.partial(
        pl.kernel,
        mesh=mesh,
        out_type=jax.ShapeDtypeStruct((x_rows, 128), jnp.float32),
        scratch_types=[
            pltpu.VMEM((32, 128), jnp.float32),
            pltpu.VMEM((32, 128), jnp.float32),
            pltpu.VMEM((32, 128), jnp.float32),
            pltpu.VMEM((32, 128), jnp.float32),
            pltpu.SemaphoreType.DMA,
            pltpu.SemaphoreType.DMA,
            pltpu.SemaphoreType.DMA,
            pltpu.SemaphoreType.DMA,
        ],
        compiler_params=pltpu.CompilerParams(
            needs_layout_passes=False, disable_bounds_checks=True
        ),
    )
    def repack_kernel(table_hbm, x_hbm, blk0, blk1, perm0, perm1,
                      semi0, semi1, semo0, semo1):
        wid = lax.axis_index("s") * num_cores + lax.axis_index("c")
        base_per = n_blocks // num_workers
        n_more = n_blocks - base_per * num_workers
        start_w = wid * base_per + jnp.minimum(wid, n_more)
        n_w = base_per + (wid < n_more).astype(jnp.int32)

        def slot_to_block(s):
            real = s < n_w
            dummy_ordinal = wid * (slots_per_tile - base_per) + (s - n_w)
            return jnp.where(real, start_w + s, n_blocks + dummy_ordinal), real

        def start_in(s, blk, sem):
            b, real = slot_to_block(s)
            src_col = 128 * jnp.where(real, start_w + s, 0)
            return pltpu.async_copy(
                table_hbm.at[:, pl.ds(pl.multiple_of(src_col, 128), 128)],
                blk, sem,
            )

        def wait_16k(dst, sem):
            pltpu.make_async_copy(
                table_hbm.at[:, pl.ds(0, 128)], dst, sem
            ).wait()

        def permute(blk, perm):
            iota = lax.iota(jnp.int32, _LANES)
            for r in range(32):
                for cc in range(8):
                    row_ids = iota + 16 * (cc % 2)
                    col_ids = _full16((r << 2) | (cc >> 1))
                    perm[r, pl.ds(16 * cc, _LANES)] = plsc.load_gather(
                        blk, [row_ids, col_ids]
                    )

        def start_out(s, perm, sem):
            b, _ = slot_to_block(s)
            return pltpu.async_copy(
                perm,
                x_hbm.at[pl.ds(pl.multiple_of(32 * b, 32), 32), :],
                sem,
            )

        start_in(0, blk0, semi0)

        def pair_body(p, carry):
            s0 = 2 * p
            start_in(s0 + 1, blk1, semi1)
            wait_16k(blk0, semi0)

            @pl.when(p > 0)
            def _():
                wait_16k(perm0, semo0)

            permute(blk0, perm0)
            start_out(s0, perm0, semo0)

            @pl.when(p < slots_per_tile // 2 - 1)
            def _():
                start_in(s0 + 2, blk0, semi0)

            wait_16k(blk1, semi1)

            @pl.when(p > 0)
            def _():
                wait_16k(perm1, semo1)

            permute(blk1, perm1)
            start_out(s0 + 1, perm1, semo1)
            return carry

        lax.fori_loop(0, slots_per_tile // 2, pair_body, 0)
        wait_16k(perm0, semo0)
        wait_16k(perm1, semo1)

    return repack_kernel, x_rows


def _make_sc_gather(vocab, dim, batch, x_rows):
    info = plsc.get_sparse_core_info()
    num_cores, num_subcores = info.num_cores, info.num_subcores
    num_workers = num_cores * num_subcores
    assert batch % (8 * num_workers) == 0
    b_per_w = batch // num_workers
    mesh = plsc.VectorSubcoreMesh(core_axis_name="c", subcore_axis_name="s")

    @functools.partial(
        pl.kernel,
        mesh=mesh,
        out_type=jax.ShapeDtypeStruct((dim, batch), jnp.float32),
        scratch_types=[
            pltpu.VMEM((b_per_w,), jnp.int32),
            pltpu.VMEM((b_per_w,), jnp.int32),
            pltpu.VMEM((b_per_w, 128), jnp.float32),
            pltpu.VMEM((dim, b_per_w), jnp.float32),
            pltpu.SemaphoreType.DMA,
        ],
        compiler_params=pltpu.CompilerParams(needs_layout_passes=False),
    )
    def gather_kernel(idx_hbm, packed_hbm, out_hbm, idx_v, idx4_v, rows_v,
                      out_v, sem):
        wid = lax.axis_index("s") * num_cores + lax.axis_index("c")
        base = wid * b_per_w
        pltpu.sync_copy(idx_hbm.at[pl.ds(base, b_per_w)], idx_v)

        def idx4_body(c, _):
            v = idx_v[pl.ds(c * _LANES, _LANES)]
            idx4_v[pl.ds(c * _LANES, _LANES)] = lax.shift_right_logical(v, 2)
            return _

        lax.fori_loop(0, b_per_w // _LANES, idx4_body, 0, unroll=4)

        pltpu.async_copy(packed_hbm.at[idx4_v], rows_v, sem).wait()

        iota = lax.iota(jnp.int32, _LANES)

        def extract_body(jc, _):
            row_ids = jc * _LANES + iota
            q = lax.bitwise_and(idx_v[pl.ds(jc * _LANES, _LANES)], 3)
            col_base = lax.shift_left(q, 5)
            for e in range(dim):
                out_v[e, pl.ds(jc * _LANES, _LANES)] = plsc.load_gather(
                    rows_v, [row_ids, col_base + e]
                )
            return _

        lax.fori_loop(0, b_per_w // _LANES, extract_body, 0)

        pltpu.sync_copy(out_v, out_hbm.at[:, pl.ds(base, b_per_w)])

    return gather_kernel


def kernel(node_idx, table):
    batch = node_idx.shape[0]
    vocab, dim = table.shape
    repack, x_rows = _make_tc_repack(vocab, dim)
    gather = _make_sc_gather(vocab, dim, batch, x_rows)
    packed = repack(jnp.swapaxes(table, 0, 1))
    out_t = gather(node_idx.astype(jnp.int32), packed)
    return jnp.swapaxes(out_t, 0, 1)


# speedup vs baseline: 2.3776x; 2.3776x over previous
import functools

import jax
import jax.numpy as jnp
from jax import lax
from jax.experimental import pallas as pl
from jax.experimental.pallas import tpu as pltpu
from jax.experimental.pallas import tpu_sc as plsc


def _make_sc_gather(vocab, dim, batch):
    info = plsc.get_sparse_core_info()
    num_cores, num_subcores = info.num_cores, info.num_subcores
    num_workers = num_cores * num_subcores
    assert batch % (8 * num_workers) == 0
    b_per_w = batch // num_workers
    mesh = plsc.VectorSubcoreMesh(core_axis_name="c", subcore_axis_name="s")

    @functools.partial(
        pl.kernel,
        mesh=mesh,
        out_type=jax.ShapeDtypeStruct((batch, dim), jnp.float32),
        scratch_types=[
            pltpu.VMEM((b_per_w,), jnp.int32),
            pltpu.VMEM((b_per_w, dim), jnp.float32),
            pltpu.SemaphoreType.DMA,
        ],
        compiler_params=pltpu.CompilerParams(use_tc_tiling_on_sc=False),
    )
    def gather_kernel(idx_hbm, table_hbm, out_hbm, idx_v, rows_v, sem):
        wid = lax.axis_index("s") * num_cores + lax.axis_index("c")
        base = wid * b_per_w
        pltpu.sync_copy(idx_hbm.at[pl.ds(base, b_per_w)], idx_v)
        pltpu.async_copy(table_hbm.at[idx_v], rows_v, sem).wait()
        pltpu.sync_copy(rows_v, out_hbm.at[pl.ds(base, b_per_w)])

    return gather_kernel


def kernel(node_idx, table):
    batch = node_idx.shape[0]
    vocab, dim = table.shape
    gather = _make_sc_gather(vocab, dim, batch)
    return gather(node_idx.astype(jnp.int32), table)


# speedup vs baseline: 2.4158x; 1.0161x over previous
import functools

import jax
import jax.numpy as jnp
from jax import lax
from jax.experimental import pallas as pl
from jax.experimental.pallas import tpu as pltpu
from jax.experimental.pallas import tpu_sc as plsc

_LANES = 16


def _make_sc_gather(vocab, dim, batch):
    info = plsc.get_sparse_core_info()
    num_cores, num_subcores = info.num_cores, info.num_subcores
    num_workers = num_cores * num_subcores
    assert batch % (8 * num_workers) == 0
    b_per_w = batch // num_workers
    mesh = plsc.VectorSubcoreMesh(core_axis_name="c", subcore_axis_name="s")

    @functools.partial(
        pl.kernel,
        mesh=mesh,
        out_type=jax.ShapeDtypeStruct((dim, batch), jnp.float32),
        scratch_types=[
            pltpu.VMEM((b_per_w,), jnp.int32),
            pltpu.VMEM((b_per_w, 128), jnp.float32),
            pltpu.VMEM((dim, b_per_w), jnp.float32),
            pltpu.SemaphoreType.DMA,
        ],
        compiler_params=pltpu.CompilerParams(needs_layout_passes=False),
    )
    def gather_kernel(idx_hbm, padded_hbm, out_hbm, idx_v, rows_v, out_v, sem):
        wid = lax.axis_index("s") * num_cores + lax.axis_index("c")
        base = wid * b_per_w
        pltpu.sync_copy(idx_hbm.at[pl.ds(base, b_per_w)], idx_v)

        pltpu.async_copy(padded_hbm.at[idx_v], rows_v, sem).wait()

        iota = lax.iota(jnp.int32, _LANES)

        def extract_body(jc, _):
            row_ids = jc * _LANES + iota
            for e in range(dim):
                out_v[e, pl.ds(jc * _LANES, _LANES)] = plsc.load_gather(
                    rows_v, [row_ids, jnp.full((_LANES,), e, jnp.int32)]
                )
            return _

        lax.fori_loop(0, b_per_w // _LANES, extract_body, 0)

        pltpu.sync_copy(out_v, out_hbm.at[:, pl.ds(base, b_per_w)])

    return gather_kernel


def kernel(node_idx, table):
    batch = node_idx.shape[0]
    vocab, dim = table.shape
    gather = _make_sc_gather(vocab, dim, batch)
    padded = jnp.pad(table, ((0, 0), (0, 128 - dim)))
    out_t = gather(node_idx.astype(jnp.int32), padded)
    return jnp.swapaxes(out_t, 0, 1)
